# Initial kernel scaffold; baseline (speedup 1.0000x reference)
import jax
import jax.numpy as jnp
from jax import lax
from jax.experimental import pallas as pl
from jax.experimental.pallas import tpu as pltpu
from jax.experimental.pallas import tpu_sc as plsc

N = 50000
E = 1600000
T = 16
COUL = 8990000000.0 * 1.602e-19 * 1.602e-19 / 1e-10 / 4.184 / 1000.0 * 6.022e+23

NW = 32
SUB = 128
SUBS_PER_CHUNK = 16
CHUNK = SUB * SUBS_PER_CHUNK
EP = 1638400
ROWS = EP // SUB
ROWS_PER_W = ROWS // NW
CHUNKS = ROWS_PER_W // SUBS_PER_CHUNK
NPAD = 51200
ZW = NPAD // 16


def _compute_sub(ri, rj, mk_v, s, eps_v, sig_v, vout):
    for v in range(8):
        lane0 = v * 16
        rid = lax.iota(jnp.int32, 16) + lane0

        def comp(rows, col):
            return plsc.load_gather(rows, [rid, jnp.full((16,), col, jnp.int32)])

        xi, yi, zi = comp(ri, 0), comp(ri, 1), comp(ri, 2)
        xj, yj, zj = comp(rj, 0), comp(rj, 1), comp(rj, 2)
        qi, qj = comp(ri, 3), comp(rj, 3)
        ti = plsc.bitcast(comp(ri, 4), jnp.int32)
        tj = plsc.bitcast(comp(rj, 4), jnp.int32)

        dx = xj - xi
        dy = yj - yi
        dz = zj - zi
        d2 = dx * dx + dy * dy + dz * dz + 1e-6

        bits = plsc.bitcast(d2, jnp.int32)
        y = plsc.bitcast(0x5F3759DF - lax.shift_right_logical(bits, 1), jnp.float32)
        h = 0.5 * d2
        for _ in range(3):
            y = y * (1.5 - h * y * y)

        code = ti * T + tj
        eps_ij = plsc.load_gather(eps_v, [code])
        sig_ij = plsc.load_gather(sig_v, [code])

        inv_d2 = y * y
        s2 = sig_ij * sig_ij * inv_d2
        sr6 = s2 * s2 * s2
        e_lj = 4.0 * eps_ij * (sr6 * sr6 - sr6)
        e_coul = COUL * qi * qj * y
        mk = mk_v[s, pl.ds(lane0, 16)]
        vout[pl.ds(lane0, 16)] = (0.5 * mk) * (e_lj + e_coul)


def _sc_body(tbl, i2, j2, m2, eps_t, sig_t, out,
             ii_v, jj_v, mk_v, ri_a, rj_a, ri_b, rj_b, v_a, v_b,
             eps_v, sig_v, zb, acc, semg, sems):
    cid = lax.axis_index("c")
    sid = lax.axis_index("s")
    wid = cid * 16 + sid

    pltpu.sync_copy(eps_t, eps_v)
    pltpu.sync_copy(sig_t, sig_v)

    zeros16 = jnp.zeros((16,), jnp.float32)

    def zset(v, _):
        zb[pl.ds(v * 16, 16)] = zeros16
        return 0

    lax.fori_loop(0, ZW // 16, zset, 0)
    pltpu.sync_copy(zb, acc.at[pl.ds(sid * ZW, ZW)])
    plsc.subcore_barrier()

    rbase = wid * ROWS_PER_W

    def chunk_body(g, _):
        r0 = rbase + g * SUBS_PER_CHUNK
        pltpu.sync_copy(i2.at[pl.ds(r0, SUBS_PER_CHUNK)], ii_v)
        pltpu.sync_copy(j2.at[pl.ds(r0, SUBS_PER_CHUNK)], jj_v)
        pltpu.sync_copy(m2.at[pl.ds(r0, SUBS_PER_CHUNK)], mk_v)

        def pair_body(p, _):
            s_a = 2 * p
            s_b = 2 * p + 1
            g_ai = pltpu.async_copy(tbl.at[ii_v.at[s_a]], ri_a, semg)
            g_aj = pltpu.async_copy(tbl.at[jj_v.at[s_a]], rj_a, semg)
            g_bi = pltpu.async_copy(tbl.at[ii_v.at[s_b]], ri_b, semg)
            g_bj = pltpu.async_copy(tbl.at[jj_v.at[s_b]], rj_b, semg)

            g_ai.wait()
            g_aj.wait()
            _compute_sub(ri_a, rj_a, mk_v, s_a, eps_v, sig_v, v_a)
            sc_ai = pltpu.async_copy(v_a, acc.at[ii_v.at[s_a]], sems, add=True)
            sc_aj = pltpu.async_copy(v_a, acc.at[jj_v.at[s_a]], sems, add=True)

            g_bi.wait()
            g_bj.wait()
            _compute_sub(ri_b, rj_b, mk_v, s_b, eps_v, sig_v, v_b)
            sc_bi = pltpu.async_copy(v_b, acc.at[ii_v.at[s_b]], sems, add=True)
            sc_bj = pltpu.async_copy(v_b, acc.at[jj_v.at[s_b]], sems, add=True)

            sc_ai.wait()
            sc_aj.wait()
            sc_bi.wait()
            sc_bj.wait()
            return 0

        lax.fori_loop(0, SUBS_PER_CHUNK // 2, pair_body, 0)
        return 0

    lax.fori_loop(0, CHUNKS, chunk_body, 0)

    plsc.subcore_barrier()
    pltpu.sync_copy(acc.at[pl.ds(sid * ZW, ZW)], out.at[cid, pl.ds(sid * ZW, ZW)])


@jax.jit
def _sc_call(tbl, i2, j2, m2, eps_t, sig_t):
    mesh = plsc.VectorSubcoreMesh(core_axis_name="c", subcore_axis_name="s")
    return pl.kernel(
        _sc_body,
        out_type=jax.ShapeDtypeStruct((2, NPAD), jnp.float32),
        mesh=mesh,
        scratch_types=[
            pltpu.VMEM((SUBS_PER_CHUNK, SUB), jnp.int32),
            pltpu.VMEM((SUBS_PER_CHUNK, SUB), jnp.int32),
            pltpu.VMEM((SUBS_PER_CHUNK, SUB), jnp.float32),
            pltpu.VMEM((SUB, 16), jnp.float32),
            pltpu.VMEM((SUB, 16), jnp.float32),
            pltpu.VMEM((SUB, 16), jnp.float32),
            pltpu.VMEM((SUB, 16), jnp.float32),
            pltpu.VMEM((SUB,), jnp.float32),
            pltpu.VMEM((SUB,), jnp.float32),
            pltpu.VMEM((T * T,), jnp.float32),
            pltpu.VMEM((T * T,), jnp.float32),
            pltpu.VMEM((ZW,), jnp.float32),
            pltpu.VMEM_SHARED((NPAD,), jnp.float32),
            pltpu.SemaphoreType.DMA,
            pltpu.SemaphoreType.DMA,
        ],
    )(tbl, i2, j2, m2, eps_t, sig_t)


def _tc_sum_body(a_ref, o_ref):
    o_ref[...] = a_ref[0] + a_ref[1]


@jax.jit
def _tc_sum(parts):
    return pl.pallas_call(
        _tc_sum_body,
        out_shape=jax.ShapeDtypeStruct((NPAD // 128, 128), jnp.float32),
    )(parts)


def kernel(pos, atom_charge, epsilon, sigma, sb_mask_e, edge_index, atom_type):
    ei = edge_index.astype(jnp.int32)
    pad = EP - E
    i_p = jnp.concatenate([ei[0], jnp.zeros((pad,), jnp.int32)])
    j_p = jnp.concatenate([ei[1], jnp.ones((pad,), jnp.int32)])
    m_p = jnp.concatenate([sb_mask_e.astype(jnp.float32),
                           jnp.zeros((pad,), jnp.float32)])
    i2 = i_p.reshape(ROWS, SUB)
    j2 = j_p.reshape(ROWS, SUB)
    m2 = m_p.reshape(ROWS, SUB)

    tbits = lax.bitcast_convert_type(atom_type.astype(jnp.int32), jnp.float32)
    tbl = jnp.concatenate(
        [pos.astype(jnp.float32),
         atom_charge.astype(jnp.float32)[:, None],
         tbits[:, None],
         jnp.zeros((N, 11), jnp.float32)], axis=1)

    eps_t = epsilon.astype(jnp.float32).reshape(T * T)
    sig_t = sigma.astype(jnp.float32).reshape(T * T)

    parts = _sc_call(tbl, i2, j2, m2, eps_t, sig_t)
    return _tc_sum(parts.reshape(2, NPAD // 128, 128)).reshape(NPAD)[:N]


# speedup vs baseline: 81.4953x; 81.4953x over previous
import jax
import jax.numpy as jnp
from jax import lax
from jax.experimental import pallas as pl
from jax.experimental.pallas import tpu as pltpu
from jax.experimental.pallas import tpu_sc as plsc

N = 50000
E = 1600000
T = 16
COUL = 8990000000.0 * 1.602e-19 * 1.602e-19 / 1e-10 / 4.184 / 1000.0 * 6.022e+23

NW = 32
SUB = 128
SUBS_PER_CHUNK = 16
CHUNK = SUB * SUBS_PER_CHUNK
EP = 1638400
ROWS = EP // SUB
ROWS_PER_W = ROWS // NW
CHUNKS = ROWS_PER_W // SUBS_PER_CHUNK
NPAD = 51200
ZW = NPAD // 16


def _compute_sub(ri, rj, mk_v, s, eps_v, sig_v, vout):
    for v in range(8):
        lane0 = v * 16
        rid = lax.iota(jnp.int32, 16) + lane0

        def comp(rows, col):
            return plsc.load_gather(rows, [rid, jnp.full((16,), col, jnp.int32)])

        xi, yi, zi = comp(ri, 0), comp(ri, 1), comp(ri, 2)
        xj, yj, zj = comp(rj, 0), comp(rj, 1), comp(rj, 2)
        qi, qj = comp(ri, 3), comp(rj, 3)
        ti = lax.convert_element_type(comp(ri, 4), jnp.int32)
        tj = lax.convert_element_type(comp(rj, 4), jnp.int32)

        dx = xj - xi
        dy = yj - yi
        dz = zj - zi
        d2 = dx * dx + dy * dy + dz * dz + 1e-6

        bits = plsc.bitcast(d2, jnp.int32)
        y = plsc.bitcast(0x5F3759DF - lax.shift_right_logical(bits, 1), jnp.float32)
        h = 0.5 * d2
        for _ in range(3):
            y = y * (1.5 - h * y * y)

        code = ti * T + tj
        eps_ij = plsc.load_gather(eps_v, [code])
        sig_ij = plsc.load_gather(sig_v, [code])

        inv_d2 = y * y
        s2 = sig_ij * sig_ij * inv_d2
        sr6 = s2 * s2 * s2
        e_lj = 4.0 * eps_ij * (sr6 * sr6 - sr6)
        e_coul = COUL * qi * qj * y
        mk = mk_v[s, pl.ds(lane0, 16)]
        vout[pl.ds(lane0, 16)] = (0.5 * mk) * (e_lj + e_coul)


def _sc_body(tbl, i2, j2, m2, eps_t, sig_t, out,
             ii_v, jj_v, mk_v, ri_a, rj_a, ri_b, rj_b, v_a, v_b,
             eps_v, sig_v, zb, acc, semg, sems):
    cid = lax.axis_index("c")
    sid = lax.axis_index("s")
    wid = cid * 16 + sid

    pltpu.sync_copy(eps_t, eps_v)
    pltpu.sync_copy(sig_t, sig_v)

    zeros16 = jnp.zeros((16,), jnp.float32)

    def zset(v, _):
        zb[pl.ds(v * 16, 16)] = zeros16
        return 0

    lax.fori_loop(0, ZW // 16, zset, 0)
    pltpu.sync_copy(zb, acc.at[pl.ds(sid * ZW, ZW)])
    plsc.subcore_barrier()

    rbase = wid * ROWS_PER_W

    def chunk_body(g, _):
        r0 = rbase + g * SUBS_PER_CHUNK
        pltpu.sync_copy(i2.at[pl.ds(r0, SUBS_PER_CHUNK)], ii_v)
        pltpu.sync_copy(j2.at[pl.ds(r0, SUBS_PER_CHUNK)], jj_v)
        pltpu.sync_copy(m2.at[pl.ds(r0, SUBS_PER_CHUNK)], mk_v)

        def pair_body(p, _):
            s_a = 2 * p
            s_b = 2 * p + 1
            g_ai = pltpu.async_copy(tbl.at[ii_v.at[s_a]], ri_a, semg)
            g_aj = pltpu.async_copy(tbl.at[jj_v.at[s_a]], rj_a, semg)
            g_bi = pltpu.async_copy(tbl.at[ii_v.at[s_b]], ri_b, semg)
            g_bj = pltpu.async_copy(tbl.at[jj_v.at[s_b]], rj_b, semg)

            g_ai.wait()
            g_aj.wait()
            _compute_sub(ri_a, rj_a, mk_v, s_a, eps_v, sig_v, v_a)
            sc_ai = pltpu.async_copy(v_a, acc.at[ii_v.at[s_a]], sems, add=True)
            sc_aj = pltpu.async_copy(v_a, acc.at[jj_v.at[s_a]], sems, add=True)

            g_bi.wait()
            g_bj.wait()
            _compute_sub(ri_b, rj_b, mk_v, s_b, eps_v, sig_v, v_b)
            sc_bi = pltpu.async_copy(v_b, acc.at[ii_v.at[s_b]], sems, add=True)
            sc_bj = pltpu.async_copy(v_b, acc.at[jj_v.at[s_b]], sems, add=True)

            sc_ai.wait()
            sc_aj.wait()
            sc_bi.wait()
            sc_bj.wait()
            return 0

        lax.fori_loop(0, SUBS_PER_CHUNK // 2, pair_body, 0)
        return 0

    lax.fori_loop(0, CHUNKS, chunk_body, 0)

    plsc.subcore_barrier()
    pltpu.sync_copy(acc.at[pl.ds(sid * ZW, ZW)], out.at[cid, pl.ds(sid * ZW, ZW)])


@jax.jit
def _sc_call(tbl, i2, j2, m2, eps_t, sig_t):
    mesh = plsc.VectorSubcoreMesh(core_axis_name="c", subcore_axis_name="s")
    return pl.kernel(
        _sc_body,
        out_type=jax.ShapeDtypeStruct((2, NPAD), jnp.float32),
        mesh=mesh,
        compiler_params=pltpu.CompilerParams(
            needs_layout_passes=False, use_tc_tiling_on_sc=False),
        scratch_types=[
            pltpu.VMEM((SUBS_PER_CHUNK, SUB), jnp.int32),
            pltpu.VMEM((SUBS_PER_CHUNK, SUB), jnp.int32),
            pltpu.VMEM((SUBS_PER_CHUNK, SUB), jnp.float32),
            pltpu.VMEM((SUB, 16), jnp.float32),
            pltpu.VMEM((SUB, 16), jnp.float32),
            pltpu.VMEM((SUB, 16), jnp.float32),
            pltpu.VMEM((SUB, 16), jnp.float32),
            pltpu.VMEM((SUB,), jnp.float32),
            pltpu.VMEM((SUB,), jnp.float32),
            pltpu.VMEM((T * T,), jnp.float32),
            pltpu.VMEM((T * T,), jnp.float32),
            pltpu.VMEM((ZW,), jnp.float32),
            pltpu.VMEM_SHARED((NPAD,), jnp.float32),
            pltpu.SemaphoreType.DMA,
            pltpu.SemaphoreType.DMA,
        ],
    )(tbl, i2, j2, m2, eps_t, sig_t)


def _tc_sum_body(a_ref, o_ref):
    o_ref[...] = a_ref[0] + a_ref[1]


@jax.jit
def _tc_sum(parts):
    return pl.pallas_call(
        _tc_sum_body,
        out_shape=jax.ShapeDtypeStruct((NPAD // 128, 128), jnp.float32),
    )(parts)


def kernel(pos, atom_charge, epsilon, sigma, sb_mask_e, edge_index, atom_type):
    ei = edge_index.astype(jnp.int32)
    pad = EP - E
    i_p = jnp.concatenate([ei[0], jnp.zeros((pad,), jnp.int32)])
    j_p = jnp.concatenate([ei[1], jnp.ones((pad,), jnp.int32)])
    m_p = jnp.concatenate([sb_mask_e.astype(jnp.float32),
                           jnp.zeros((pad,), jnp.float32)])
    i2 = i_p.reshape(ROWS, SUB)
    j2 = j_p.reshape(ROWS, SUB)
    m2 = m_p.reshape(ROWS, SUB)

    tflt = atom_type.astype(jnp.int32).astype(jnp.float32)
    tbl = jnp.concatenate(
        [pos.astype(jnp.float32),
         atom_charge.astype(jnp.float32)[:, None],
         tflt[:, None],
         jnp.zeros((N, 11), jnp.float32)], axis=1)

    eps_t = epsilon.astype(jnp.float32).reshape(T * T)
    sig_t = sigma.astype(jnp.float32).reshape(T * T)

    parts = _sc_call(tbl, i2, j2, m2, eps_t, sig_t)
    return _tc_sum(parts.reshape(2, NPAD // 128, 128)).reshape(NPAD)[:N]


# speedup vs baseline: 120.9243x; 1.4838x over previous
import jax
import jax.numpy as jnp
from jax import lax
from jax.experimental import pallas as pl
from jax.experimental.pallas import tpu as pltpu
from jax.experimental.pallas import tpu_sc as plsc

N = 50000
E = 1600000
T = 16
COUL = 8990000000.0 * 1.602e-19 * 1.602e-19 / 1e-10 / 4.184 / 1000.0 * 6.022e+23

NW = 32
SUB = 1024
SUBS_PER_CHUNK = 10
EP = 1638400
ROWS = EP // SUB
ROWS_PER_W = ROWS // NW
CHUNKS = ROWS_PER_W // SUBS_PER_CHUNK
VPS = SUB // 16
UNROLL = 4
NPAD = 51200
ZW = NPAD // 16


def _compute_sub(ri, rj, mk_v, s, eps_v, sig_v, vout):
    def vblock(u, _):
        for w in range(UNROLL):
            lane0 = (u * UNROLL + w) * 16
            rid = lax.iota(jnp.int32, 16) + lane0

            def comp(rows, col):
                return plsc.load_gather(
                    rows, [rid, jnp.full((16,), col, jnp.int32)])

            xi, yi, zi = comp(ri, 0), comp(ri, 1), comp(ri, 2)
            xj, yj, zj = comp(rj, 0), comp(rj, 1), comp(rj, 2)
            qi, qj = comp(ri, 3), comp(rj, 3)
            ti = lax.convert_element_type(comp(ri, 4), jnp.int32)
            tj = lax.convert_element_type(comp(rj, 4), jnp.int32)

            dx = xj - xi
            dy = yj - yi
            dz = zj - zi
            d2 = dx * dx + dy * dy + dz * dz + 1e-6

            bits = plsc.bitcast(d2, jnp.int32)
            y = plsc.bitcast(
                0x5F3759DF - lax.shift_right_logical(bits, 1), jnp.float32)
            h = 0.5 * d2
            for _ in range(3):
                y = y * (1.5 - h * y * y)

            code = ti * T + tj
            eps_ij = plsc.load_gather(eps_v, [code])
            sig_ij = plsc.load_gather(sig_v, [code])

            inv_d2 = y * y
            s2 = sig_ij * sig_ij * inv_d2
            sr6 = s2 * s2 * s2
            e_lj = 4.0 * eps_ij * (sr6 * sr6 - sr6)
            e_coul = COUL * qi * qj * y
            mk = mk_v[s, pl.ds(lane0, 16)]
            vout[pl.ds(lane0, 16)] = (0.5 * mk) * (e_lj + e_coul)
        return 0

    lax.fori_loop(0, VPS // UNROLL, vblock, 0)


def _sc_body(tbl, i2, j2, m2, eps_t, sig_t, out,
             ii_v, jj_v, mk_v, ri_a, rj_a, ri_b, rj_b, v_a, v_b,
             eps_v, sig_v, zb, acc, semg, sems):
    cid = lax.axis_index("c")
    sid = lax.axis_index("s")
    wid = cid * 16 + sid

    pltpu.sync_copy(eps_t, eps_v)
    pltpu.sync_copy(sig_t, sig_v)

    zeros16 = jnp.zeros((16,), jnp.float32)

    def zset(v, _):
        zb[pl.ds(v * 16, 16)] = zeros16
        return 0

    lax.fori_loop(0, ZW // 16, zset, 0)
    pltpu.sync_copy(zb, acc.at[pl.ds(sid * ZW, ZW)])
    plsc.subcore_barrier()

    rbase = wid * ROWS_PER_W

    def drain_scatters():
        pltpu.make_async_copy(v_a, acc.at[ii_v.at[0]], sems).wait()
        pltpu.make_async_copy(v_a, acc.at[jj_v.at[0]], sems).wait()
        pltpu.make_async_copy(v_b, acc.at[ii_v.at[0]], sems).wait()
        pltpu.make_async_copy(v_b, acc.at[jj_v.at[0]], sems).wait()

    def chunk_body(g, _):
        @pl.when(g > 0)
        def _():
            drain_scatters()

        r0 = rbase + g * SUBS_PER_CHUNK
        pltpu.sync_copy(i2.at[pl.ds(r0, SUBS_PER_CHUNK)], ii_v)
        pltpu.sync_copy(j2.at[pl.ds(r0, SUBS_PER_CHUNK)], jj_v)
        pltpu.sync_copy(m2.at[pl.ds(r0, SUBS_PER_CHUNK)], mk_v)

        def pair_body(p, _):
            s_a = 2 * p
            s_b = 2 * p + 1
            g_ai = pltpu.async_copy(tbl.at[ii_v.at[s_a]], ri_a, semg)
            g_aj = pltpu.async_copy(tbl.at[jj_v.at[s_a]], rj_a, semg)
            g_bi = pltpu.async_copy(tbl.at[ii_v.at[s_b]], ri_b, semg)
            g_bj = pltpu.async_copy(tbl.at[jj_v.at[s_b]], rj_b, semg)

            @pl.when(p > 0)
            def _():
                drain_scatters()

            g_ai.wait()
            g_aj.wait()
            _compute_sub(ri_a, rj_a, mk_v, s_a, eps_v, sig_v, v_a)
            pltpu.async_copy(v_a, acc.at[ii_v.at[s_a]], sems, add=True)
            pltpu.async_copy(v_a, acc.at[jj_v.at[s_a]], sems, add=True)

            g_bi.wait()
            g_bj.wait()
            _compute_sub(ri_b, rj_b, mk_v, s_b, eps_v, sig_v, v_b)
            pltpu.async_copy(v_b, acc.at[ii_v.at[s_b]], sems, add=True)
            pltpu.async_copy(v_b, acc.at[jj_v.at[s_b]], sems, add=True)
            return 0

        lax.fori_loop(0, SUBS_PER_CHUNK // 2, pair_body, 0)
        return 0

    lax.fori_loop(0, CHUNKS, chunk_body, 0)

    pltpu.make_async_copy(v_a, acc.at[ii_v.at[0]], sems).wait()
    pltpu.make_async_copy(v_a, acc.at[jj_v.at[0]], sems).wait()
    pltpu.make_async_copy(v_b, acc.at[ii_v.at[0]], sems).wait()
    pltpu.make_async_copy(v_b, acc.at[jj_v.at[0]], sems).wait()

    plsc.subcore_barrier()
    pltpu.sync_copy(acc.at[pl.ds(sid * ZW, ZW)], out.at[cid, pl.ds(sid * ZW, ZW)])


@jax.jit
def _sc_call(tbl, i2, j2, m2, eps_t, sig_t):
    mesh = plsc.VectorSubcoreMesh(core_axis_name="c", subcore_axis_name="s")
    return pl.kernel(
        _sc_body,
        out_type=jax.ShapeDtypeStruct((2, NPAD), jnp.float32),
        mesh=mesh,
        compiler_params=pltpu.CompilerParams(
            needs_layout_passes=False, use_tc_tiling_on_sc=False),
        scratch_types=[
            pltpu.VMEM((SUBS_PER_CHUNK, SUB), jnp.int32),
            pltpu.VMEM((SUBS_PER_CHUNK, SUB), jnp.int32),
            pltpu.VMEM((SUBS_PER_CHUNK, SUB), jnp.float32),
            pltpu.VMEM((SUB, 8), jnp.float32),
            pltpu.VMEM((SUB, 8), jnp.float32),
            pltpu.VMEM((SUB, 8), jnp.float32),
            pltpu.VMEM((SUB, 8), jnp.float32),
            pltpu.VMEM((SUB,), jnp.float32),
            pltpu.VMEM((SUB,), jnp.float32),
            pltpu.VMEM((T * T,), jnp.float32),
            pltpu.VMEM((T * T,), jnp.float32),
            pltpu.VMEM((ZW,), jnp.float32),
            pltpu.VMEM_SHARED((NPAD,), jnp.float32),
            pltpu.SemaphoreType.DMA,
            pltpu.SemaphoreType.DMA,
        ],
    )(tbl, i2, j2, m2, eps_t, sig_t)


def _tc_sum_body(a_ref, o_ref):
    o_ref[...] = a_ref[0] + a_ref[1]


@jax.jit
def _tc_sum(parts):
    return pl.pallas_call(
        _tc_sum_body,
        out_shape=jax.ShapeDtypeStruct((NPAD // 128, 128), jnp.float32),
    )(parts)


def kernel(pos, atom_charge, epsilon, sigma, sb_mask_e, edge_index, atom_type):
    ei = edge_index.astype(jnp.int32)
    pad = EP - E
    i_p = jnp.concatenate([ei[0], jnp.zeros((pad,), jnp.int32)])
    j_p = jnp.concatenate([ei[1], jnp.ones((pad,), jnp.int32)])
    m_p = jnp.concatenate([sb_mask_e.astype(jnp.float32),
                           jnp.zeros((pad,), jnp.float32)])
    i2 = i_p.reshape(ROWS, SUB)
    j2 = j_p.reshape(ROWS, SUB)
    m2 = m_p.reshape(ROWS, SUB)

    tflt = atom_type.astype(jnp.int32).astype(jnp.float32)
    tbl = jnp.concatenate(
        [pos.astype(jnp.float32),
         atom_charge.astype(jnp.float32)[:, None],
         tflt[:, None],
         jnp.zeros((N, 3), jnp.float32)], axis=1)

    eps_t = epsilon.astype(jnp.float32).reshape(T * T)
    sig_t = sigma.astype(jnp.float32).reshape(T * T)

    parts = _sc_call(tbl, i2, j2, m2, eps_t, sig_t)
    return _tc_sum(parts.reshape(2, NPAD // 128, 128)).reshape(NPAD)[:N]
